# baseline (speedup 1.0000x reference)
import functools

import jax
import jax.numpy as jnp
from jax import lax
from jax.experimental import pallas as pl
from jax.experimental.pallas import tpu as pltpu
from jax.experimental.pallas import tpu_sc as plsc

_NUM_CODES = 512
_DIM = 64
_T = 1024


_BPS = 8


def _tc_body(xt_ref, c_ref, idx_ref, loss_ref, tbl_ref):
    i = pl.program_id(0)
    c = c_ref[...]
    cb = (-2.0 * c).astype(jnp.bfloat16)
    cn = jnp.sum(c * c, axis=1, keepdims=True)

    @pl.when(i == 0)
    def _():
        rounded = cb.astype(jnp.float32) * -0.5
        tbl_ref[...] = lax.transpose(rounded, (1, 0))

    iota8 = lax.broadcasted_iota(jnp.int32, (8, 1024), 0)
    s = None
    for k in range(_BPS):
        xk = xt_ref[k * _DIM:(k + 1) * _DIM, :]
        xkb = xk.astype(jnp.bfloat16)
        xc2 = lax.dot_general(cb, xkb, (((1,), (0,)), ((), ())),
                              preferred_element_type=jnp.float32)
        xn = jnp.sum(xk * xk, axis=0, keepdims=True)
        best = bidx = None
        for r in range(_NUM_CODES // 8):
            d_r = (xn + cn[r * 8:(r + 1) * 8]) + xc2[r * 8:(r + 1) * 8]
            i_r = iota8 + (r * 8)
            if best is None:
                best, bidx = d_r, i_r
            else:
                pred = d_r < best
                best = jnp.where(pred, d_r, best)
                bidx = jnp.where(pred, i_r, bidx)
        m = jnp.min(best, axis=0, keepdims=True)
        cand = jnp.where(best == m, bidx, _NUM_CODES)
        idxr = jnp.min(cand, axis=0, keepdims=True)
        idx_ref[k:k + 1, :] = idxr
        sk = jnp.sum(m, keepdims=True).reshape(1, 1)
        s = sk if s is None else s + sk

    @pl.when(i == 0)
    def _():
        loss_ref[...] = s

    @pl.when(i != 0)
    def _():
        loss_ref[...] += s


def _tc_argmin_call(xt, codebook, off=0, nbatch=None, interpret=False):
    if nbatch is None:
        nbatch = xt.shape[0] // _DIM
    nsteps = nbatch // _BPS
    return pl.pallas_call(
        _tc_body,
        grid=(nsteps,),
        in_specs=[
            pl.BlockSpec((_BPS * _DIM, 1024), lambda i: (i + off, 0)),
            pl.BlockSpec((_NUM_CODES, _DIM), lambda i: (0, 0)),
        ],
        out_specs=[
            pl.BlockSpec((_BPS, 1024), lambda i: (i, 0)),
            pl.BlockSpec((1, 1), lambda i: (0, 0)),
            pl.BlockSpec((_DIM, _NUM_CODES), lambda i: (0, 0)),
        ],
        out_shape=[
            jax.ShapeDtypeStruct((nbatch, 1024), jnp.int32),
            jax.ShapeDtypeStruct((1, 1), jnp.float32),
            jax.ShapeDtypeStruct((_DIM, _NUM_CODES), jnp.float32),
        ],
        compiler_params=pltpu.CompilerParams(
            dimension_semantics=("arbitrary",)),
        interpret=interpret,
    )(xt, codebook)


_TPW = 256
_GRP = 16


def _sc_gather_call(tableT, idx64):
    nbatch, ntok = idx64.shape
    mesh = plsc.VectorSubcoreMesh(core_axis_name="c", subcore_axis_name="s")
    ncg = ntok // _TPW
    nband = 32 // ncg
    bpb = nbatch // nband

    @functools.partial(
        pl.kernel,
        mesh=mesh,
        out_type=jax.ShapeDtypeStruct((nbatch * _DIM, ntok), jnp.float32),
        scratch_types=[
            pltpu.VMEM((_DIM, _NUM_CODES), jnp.float32),
            pltpu.VMEM((bpb, _TPW), jnp.int32),
            [pltpu.VMEM((_DIM, _TPW), jnp.float32)] * 2,
            pltpu.SemaphoreType.DMA,
            pltpu.SemaphoreType.DMA,
            [pltpu.SemaphoreType.DMA] * 2,
        ],
        compiler_params=pltpu.CompilerParams(needs_layout_passes=False),
    )
    def gather_k(table_hbm, idx_hbm, out_hbm, table_v, idx_v, q_v, tsem, isem,
                 ssems):
        wid = lax.axis_index("s") * 2 + lax.axis_index("c")
        band = wid // ncg
        cg = lax.rem(wid, ncg)
        tcopy = pltpu.async_copy(table_hbm, table_v, tsem)
        pltpu.async_copy(
            idx_hbm.at[pl.ds(band * bpb, bpb), pl.ds(cg * _TPW, _TPW)],
            idx_v, isem).wait()
        tcopy.wait()
        scatters = [None, None]
        for bb in range(bpb):
            buf = bb % 2
            if scatters[buf] is not None:
                scatters[buf].wait()

            @plsc.parallel_loop(0, _TPW // _GRP)
            def body(g, bb=bb, buf=buf):
                idx16 = idx_v[bb, pl.ds(g * _GRP, _GRP)]
                for d in range(_DIM):
                    q_v[buf][d, pl.ds(g * _GRP, _GRP)] = plsc.load_gather(
                        table_v, [jnp.full((_GRP,), d, jnp.int32), idx16])
            scatters[buf] = pltpu.async_copy(
                q_v[buf],
                out_hbm.at[pl.ds((band * bpb + bb) * _DIM, _DIM),
                           pl.ds(cg * _TPW, _TPW)],
                ssems[buf])
        scatters[0].wait()
        scatters[1].wait()

    return gather_k(tableT, idx64)


def kernel(inputs, codebook):
    xt = jnp.swapaxes(inputs, 1, 2).reshape(-1, inputs.shape[1])
    nbatch = xt.shape[0] // _DIM
    half = nbatch // 2
    idx0, loss0, tableT = _tc_argmin_call(xt, codebook, off=0, nbatch=half)
    qt0 = _sc_gather_call(tableT, idx0)
    idx1, loss1, _ = _tc_argmin_call(xt, codebook, off=half // _BPS,
                                     nbatch=half)
    qt1 = _sc_gather_call(tableT, idx1)
    qt = jnp.concatenate([qt0, qt1], axis=0)
    n_elems = nbatch * 1024 * _DIM
    loss = (loss0[0, 0] + loss1[0, 0]) * jnp.float32(1.25 / n_elems)
    nb = inputs.shape[0]
    q = jnp.swapaxes(qt.reshape(nb, _DIM, inputs.shape[1]), 1, 2)
    return loss, q


# speedup vs baseline: 1.3939x; 1.3939x over previous
import functools

import jax
import jax.numpy as jnp
from jax import lax
from jax.experimental import pallas as pl
from jax.experimental.pallas import tpu as pltpu
from jax.experimental.pallas import tpu_sc as plsc

_NUM_CODES = 512
_DIM = 64
_T = 1024


_BPS = 8


def _tc_body(xt_ref, c_ref, idx_ref, loss_ref, tbl_ref):
    i = pl.program_id(0)
    c = c_ref[...]
    cb = (-2.0 * c).astype(jnp.bfloat16)
    cn = jnp.sum(c * c, axis=1, keepdims=True)

    @pl.when(i == 0)
    def _():
        rounded = cb.astype(jnp.float32) * -0.5
        tbl_ref[...] = lax.transpose(rounded, (1, 0))

    iota8 = lax.broadcasted_iota(jnp.int32, (8, 1024), 0)
    s = None
    for k in range(_BPS):
        xk = xt_ref[k * _DIM:(k + 1) * _DIM, :]
        xkb = xk.astype(jnp.bfloat16)
        xc2 = lax.dot_general(cb, xkb, (((1,), (0,)), ((), ())),
                              preferred_element_type=jnp.float32)
        xn = jnp.sum(xk * xk, axis=0, keepdims=True)
        best = bidx = None
        for r in range(_NUM_CODES // 8):
            d_r = (xn + cn[r * 8:(r + 1) * 8]) + xc2[r * 8:(r + 1) * 8]
            i_r = iota8 + (r * 8)
            if best is None:
                best, bidx = d_r, i_r
            else:
                pred = d_r < best
                best = jnp.where(pred, d_r, best)
                bidx = jnp.where(pred, i_r, bidx)
        m = jnp.min(best, axis=0, keepdims=True)
        cand = jnp.where(best == m, bidx, _NUM_CODES)
        idxr = jnp.min(cand, axis=0, keepdims=True)
        idx_ref[k:k + 1, :] = idxr
        sk = jnp.sum(m, keepdims=True).reshape(1, 1)
        s = sk if s is None else s + sk

    @pl.when(i == 0)
    def _():
        loss_ref[...] = s

    @pl.when(i != 0)
    def _():
        loss_ref[...] += s


def _tc_argmin_call(xt, codebook, off=0, nbatch=None, interpret=False):
    if nbatch is None:
        nbatch = xt.shape[0] // _DIM
    nsteps = nbatch // _BPS
    return pl.pallas_call(
        _tc_body,
        grid=(nsteps,),
        in_specs=[
            pl.BlockSpec((_BPS * _DIM, 1024), lambda i: (i + off, 0)),
            pl.BlockSpec((_NUM_CODES, _DIM), lambda i: (0, 0)),
        ],
        out_specs=[
            pl.BlockSpec((_BPS, 1024), lambda i: (i, 0)),
            pl.BlockSpec((1, 1), lambda i: (0, 0)),
            pl.BlockSpec((_DIM, _NUM_CODES), lambda i: (0, 0)),
        ],
        out_shape=[
            jax.ShapeDtypeStruct((nbatch, 1024), jnp.int32),
            jax.ShapeDtypeStruct((1, 1), jnp.float32),
            jax.ShapeDtypeStruct((_DIM, _NUM_CODES), jnp.float32),
        ],
        compiler_params=pltpu.CompilerParams(
            dimension_semantics=("arbitrary",)),
        interpret=interpret,
    )(xt, codebook)


_TPW = 256
_GRP = 16


def _sc_gather_call(tableT, idx64):
    nbatch, ntok = idx64.shape
    mesh = plsc.VectorSubcoreMesh(core_axis_name="c", subcore_axis_name="s")
    ncg = ntok // _TPW
    nband = 32 // ncg
    bpb = nbatch // nband

    @functools.partial(
        pl.kernel,
        mesh=mesh,
        out_type=jax.ShapeDtypeStruct((nbatch * _DIM, ntok), jnp.float32),
        scratch_types=[
            pltpu.VMEM((_DIM, _NUM_CODES), jnp.float32),
            pltpu.VMEM((bpb, _TPW), jnp.int32),
            [pltpu.VMEM((_DIM, _TPW), jnp.float32)] * 2,
            pltpu.SemaphoreType.DMA,
            pltpu.SemaphoreType.DMA,
            [pltpu.SemaphoreType.DMA] * 2,
        ],
        compiler_params=pltpu.CompilerParams(needs_layout_passes=False),
    )
    def gather_k(table_hbm, idx_hbm, out_hbm, table_v, idx_v, q_v, tsem, isem,
                 ssems):
        wid = lax.axis_index("s") * 2 + lax.axis_index("c")
        band = wid // ncg
        cg = lax.rem(wid, ncg)
        tcopy = pltpu.async_copy(table_hbm, table_v, tsem)
        pltpu.async_copy(
            idx_hbm.at[pl.ds(band * bpb, bpb), pl.ds(cg * _TPW, _TPW)],
            idx_v, isem).wait()
        tcopy.wait()
        scatters = [None, None]
        for bb in range(bpb):
            buf = bb % 2
            if scatters[buf] is not None:
                scatters[buf].wait()

            def gbody(g, c, bb=bb, buf=buf):
                idx16 = idx_v[bb, pl.ds(g * _GRP, _GRP)]

                @plsc.parallel_loop(0, _DIM, unroll=8)
                def dbody(d, g=g, idx16=idx16, buf=buf):
                    q_v[buf][d, pl.ds(g * _GRP, _GRP)] = plsc.load_gather(
                        table_v, [jnp.full((_GRP,), d, jnp.int32), idx16])

                return c

            lax.fori_loop(0, _TPW // _GRP, gbody, 0)
            scatters[buf] = pltpu.async_copy(
                q_v[buf],
                out_hbm.at[pl.ds((band * bpb + bb) * _DIM, _DIM),
                           pl.ds(cg * _TPW, _TPW)],
                ssems[buf])
        scatters[0].wait()
        scatters[1].wait()

    return gather_k(tableT, idx64)


def kernel(inputs, codebook):
    xt = jnp.swapaxes(inputs, 1, 2).reshape(-1, inputs.shape[1])
    idx64, loss_sum, tableT = _tc_argmin_call(xt, codebook)
    qt = _sc_gather_call(tableT, idx64)
    n_elems = idx64.size * _DIM
    loss = loss_sum[0, 0] * jnp.float32(1.25 / n_elems)
    nb = inputs.shape[0]
    q = jnp.swapaxes(qt.reshape(nb, _DIM, inputs.shape[1]), 1, 2)
    return loss, q
